# initial kernel scaffold (unmeasured)
import jax
import jax.numpy as jnp
from jax import lax
from jax.experimental import pallas as pl
from jax.experimental.pallas import tpu as pltpu

N = 16
C = 256
D = 512
H = 1024
E_LOC = 8
PAY = 640


def _body(xsend_ref, x_ref, shW_ref, eW_ref,
          yrecv_ref, shared_ref,
          recv_ref, y_ref, ss1, rs1, ss2, rs2):
    me = lax.axis_index("i")

    barrier = pltpu.get_barrier_semaphore()
    for k in range(1, N):
        nbr = (me + k) % N
        pl.semaphore_signal(barrier, inc=1, device_id=(nbr,),
                            device_id_type=pl.DeviceIdType.MESH)
    pl.semaphore_wait(barrier, N - 1)

    send1 = []
    for k in range(1, N):
        dst = (me + k) % N
        r = pltpu.make_async_remote_copy(
            src_ref=xsend_ref.at[pl.ds(dst * C, C), :],
            dst_ref=recv_ref.at[pl.ds(me * C, C), :],
            send_sem=ss1.at[dst],
            recv_sem=rs1.at[me],
            device_id=(dst,),
            device_id_type=pl.DeviceIdType.MESH,
        )
        r.start()
        send1.append(r)

    recv_ref[pl.ds(me * C, C), :] = xsend_ref[pl.ds(me * C, C), :]

    shared_ref[...] = jnp.dot(x_ref[...], shW_ref[...],
                              preferred_element_type=jnp.float32)

    for k in range(1, N):
        s = (me + k) % N
        r = pltpu.make_async_remote_copy(
            src_ref=xsend_ref.at[pl.ds(0, C), :],
            dst_ref=recv_ref.at[pl.ds(s * C, C), :],
            send_sem=ss1.at[s],
            recv_sem=rs1.at[s],
            device_id=(s,),
            device_id_type=pl.DeviceIdType.MESH,
        )
        r.wait_recv()

    xp = recv_ref[:, :D]
    for j in range(E_LOC):
        contrib = jnp.dot(xp * recv_ref[:, D + j][:, None], eW_ref[j],
                          preferred_element_type=jnp.float32)
        if j == 0:
            y_ref[...] = contrib
        else:
            y_ref[...] = y_ref[...] + contrib

    send2 = []
    for k in range(1, N):
        m = (me + k) % N
        r = pltpu.make_async_remote_copy(
            src_ref=y_ref.at[pl.ds(m * C, C), :],
            dst_ref=yrecv_ref.at[pl.ds(me * C, C), :],
            send_sem=ss2.at[m],
            recv_sem=rs2.at[me],
            device_id=(m,),
            device_id_type=pl.DeviceIdType.MESH,
        )
        r.start()
        send2.append(r)

    yrecv_ref[pl.ds(me * C, C), :] = y_ref[pl.ds(me * C, C), :]

    for k in range(1, N):
        s = (me + k) % N
        r = pltpu.make_async_remote_copy(
            src_ref=y_ref.at[pl.ds(0, C), :],
            dst_ref=yrecv_ref.at[pl.ds(s * C, C), :],
            send_sem=ss2.at[s],
            recv_sem=rs2.at[s],
            device_id=(s,),
            device_id_type=pl.DeviceIdType.MESH,
        )
        r.wait_recv()

    for r in send1:
        r.wait_send()
    for r in send2:
        r.wait_send()


def kernel(x, router_W, route_idx, expert_W, shared_W):
    n_tok = x.shape[0]

    scores = jnp.dot(x, router_W, preferred_element_type=jnp.float32)
    scores = scores - scores.max(axis=1, keepdims=True)
    probs = jnp.exp(scores)
    probs = probs / probs.sum(axis=1, keepdims=True)
    p = jnp.take_along_axis(probs, route_idx, axis=1)[:, 0]

    e = route_idx[:, 0]
    dst = e // E_LOC
    eloc = e % E_LOC

    oh_dst = (dst[:, None] == jnp.arange(N, dtype=dst.dtype)[None, :]).astype(jnp.int32)
    j_slot = jnp.take_along_axis(jnp.cumsum(oh_dst, axis=0), dst[:, None], axis=1)[:, 0] - 1
    flat = jnp.where(j_slot < C, dst * C + j_slot, N * C)

    xp = x * p[:, None]
    oh_e = (eloc[:, None] == jnp.arange(E_LOC, dtype=eloc.dtype)[None, :]).astype(x.dtype)
    val = jnp.concatenate(
        [xp, oh_e, jnp.zeros((n_tok, PAY - D - E_LOC), x.dtype)], axis=1)
    xsend = jnp.zeros((N * C, PAY), x.dtype).at[flat].set(val, mode="drop")

    yrecv, shared = pl.pallas_call(
        _body,
        out_shape=(
            jax.ShapeDtypeStruct((N * C, H), jnp.float32),
            jax.ShapeDtypeStruct((n_tok, H), jnp.float32),
        ),
        in_specs=[
            pl.BlockSpec(memory_space=pltpu.VMEM),
            pl.BlockSpec(memory_space=pltpu.VMEM),
            pl.BlockSpec(memory_space=pltpu.VMEM),
            pl.BlockSpec(memory_space=pltpu.VMEM),
        ],
        out_specs=(
            pl.BlockSpec(memory_space=pltpu.VMEM),
            pl.BlockSpec(memory_space=pltpu.VMEM),
        ),
        scratch_shapes=[
            pltpu.VMEM((N * C, PAY), jnp.float32),
            pltpu.VMEM((N * C, H), jnp.float32),
            pltpu.SemaphoreType.DMA((N,)),
            pltpu.SemaphoreType.DMA((N,)),
            pltpu.SemaphoreType.DMA((N,)),
            pltpu.SemaphoreType.DMA((N,)),
        ],
        compiler_params=pltpu.CompilerParams(collective_id=0),
    )(xsend, x, shared_W, expert_W)

    safe = jnp.clip(flat, 0, N * C - 1)
    expert_out = jnp.where((j_slot < C)[:, None], yrecv[safe], 0.0)
    return shared + expert_out


# baseline (device time: 580326 ns/iter reference)
import jax
import jax.numpy as jnp
from jax import lax
from jax.experimental import pallas as pl
from jax.experimental.pallas import tpu as pltpu

N = 16
C = 192
D = 512
H = 1024
E_LOC = 8
PAY = 640


def _expert_chunk(chunk, eW_ref):
    xp = chunk[:, :D]
    acc = None
    for j in range(E_LOC):
        contrib = jnp.dot(xp * chunk[:, D + j][:, None], eW_ref[j],
                          preferred_element_type=jnp.float32)
        acc = contrib if acc is None else acc + contrib
    return acc


def _body(xsend_ref, eW_ref, yrecv_ref, recv_ref, ybuf_ref, ss1, rs1, ss2, rs2):
    me = lax.axis_index("i")

    barrier = pltpu.get_barrier_semaphore()
    for k in range(1, N):
        nbr = (me + k) % N
        pl.semaphore_signal(barrier, inc=1, device_id=(nbr,),
                            device_id_type=pl.DeviceIdType.MESH)
    pl.semaphore_wait(barrier, N - 1)

    send1 = []
    for k in range(1, N):
        dst = (me + k) % N
        r = pltpu.make_async_remote_copy(
            src_ref=xsend_ref.at[pl.ds(k * C, C), :],
            dst_ref=recv_ref.at[pl.ds((N - k) * C, C), :],
            send_sem=ss1.at[k],
            recv_sem=rs1.at[N - k],
            device_id=(dst,),
            device_id_type=pl.DeviceIdType.MESH,
        )
        r.start()
        send1.append(r)

    yrecv_ref[pl.ds(0, C), :] = _expert_chunk(xsend_ref[pl.ds(0, C), :], eW_ref)

    send2 = []
    for j in range(1, N):
        s = (me + j) % N
        rwait = pltpu.make_async_remote_copy(
            src_ref=xsend_ref.at[pl.ds(0, C), :],
            dst_ref=recv_ref.at[pl.ds(j * C, C), :],
            send_sem=ss1.at[j],
            recv_sem=rs1.at[j],
            device_id=(s,),
            device_id_type=pl.DeviceIdType.MESH,
        )
        rwait.wait_recv()

        slot = (j - 1) % 2
        if len(send2) >= 2:
            send2[-2].wait_send()
        ybuf_ref[slot] = _expert_chunk(recv_ref[pl.ds(j * C, C), :], eW_ref)

        r = pltpu.make_async_remote_copy(
            src_ref=ybuf_ref.at[slot],
            dst_ref=yrecv_ref.at[pl.ds((N - j) * C, C), :],
            send_sem=ss2.at[j],
            recv_sem=rs2.at[N - j],
            device_id=(s,),
            device_id_type=pl.DeviceIdType.MESH,
        )
        r.start()
        send2.append(r)

    for j in range(1, N):
        rwait = pltpu.make_async_remote_copy(
            src_ref=ybuf_ref.at[0],
            dst_ref=yrecv_ref.at[pl.ds(j * C, C), :],
            send_sem=ss2.at[j],
            recv_sem=rs2.at[j],
            device_id=((me + j) % N,),
            device_id_type=pl.DeviceIdType.MESH,
        )
        rwait.wait_recv()

    for r in send1:
        r.wait_send()
    for r in send2[-2:]:
        r.wait_send()


def kernel(x, router_W, route_idx, expert_W, shared_W):
    n_tok = x.shape[0]
    me = lax.axis_index("i")

    scores = jnp.dot(x, router_W, preferred_element_type=jnp.float32)
    scores = scores - scores.max(axis=1, keepdims=True)
    probs = jnp.exp(scores)
    probs = probs / probs.sum(axis=1, keepdims=True)
    p = jnp.take_along_axis(probs, route_idx, axis=1)[:, 0]

    e = route_idx[:, 0]
    dst = e // E_LOC
    eloc = e % E_LOC
    rel = (dst - me) % N

    oh_dst = (dst[:, None] == jnp.arange(N, dtype=dst.dtype)[None, :]).astype(jnp.int32)
    j_slot = jnp.take_along_axis(jnp.cumsum(oh_dst, axis=0), dst[:, None], axis=1)[:, 0] - 1
    flat = jnp.where(j_slot < C, rel * C + j_slot, N * C)

    xp = x * p[:, None]
    oh_e = (eloc[:, None] == jnp.arange(E_LOC, dtype=eloc.dtype)[None, :]).astype(x.dtype)
    val = jnp.concatenate(
        [xp, oh_e, jnp.zeros((n_tok, PAY - D - E_LOC), x.dtype)], axis=1)
    xsend = jnp.zeros((N * C, PAY), x.dtype).at[flat].set(val, mode="drop")

    yrecv = pl.pallas_call(
        _body,
        out_shape=jax.ShapeDtypeStruct((N * C, H), jnp.float32),
        in_specs=[
            pl.BlockSpec(memory_space=pltpu.VMEM),
            pl.BlockSpec(memory_space=pltpu.VMEM),
        ],
        out_specs=pl.BlockSpec(memory_space=pltpu.VMEM),
        scratch_shapes=[
            pltpu.VMEM((N * C, PAY), jnp.float32),
            pltpu.VMEM((2, C, H), jnp.float32),
            pltpu.SemaphoreType.DMA((N,)),
            pltpu.SemaphoreType.DMA((N,)),
            pltpu.SemaphoreType.DMA((N,)),
            pltpu.SemaphoreType.DMA((N,)),
        ],
        compiler_params=pltpu.CompilerParams(collective_id=0),
    )(xsend, expert_W)

    shared = jnp.dot(x, shared_W, preferred_element_type=jnp.float32)
    safe = jnp.clip(flat, 0, N * C - 1)
    expert_out = jnp.where((j_slot < C)[:, None], yrecv[safe], 0.0)
    return shared + expert_out


# device time: 370086 ns/iter; 1.5681x vs baseline; 1.5681x over previous
import jax
import jax.numpy as jnp
from jax import lax
from jax.experimental import pallas as pl
from jax.experimental.pallas import tpu as pltpu

N = 16
C = 192
D = 512
H = 1024
E_LOC = 8
PAY = 640
T = 2048


def _expert_chunk(chunk, eW_ref):
    xp = chunk[:, :D]
    acc = None
    for j in range(E_LOC):
        contrib = jnp.dot(xp * chunk[:, D + j][:, None], eW_ref[j],
                          preferred_element_type=jnp.float32)
        acc = contrib if acc is None else acc + contrib
    return acc


def _pack_chunk(k, flat, val):
    rows = lax.broadcasted_iota(jnp.int32, (C, T), 0) + k * C
    p_mat = (rows == flat).astype(jnp.float32)
    return jnp.dot(p_mat, val, preferred_element_type=jnp.float32)


def _body(val_ref, flat_ref, eW_ref, yrecv_ref,
          xsend_ref, recv_ref, ybuf_ref, ss1, rs1, ss2, rs2, cp_sem):
    me = lax.axis_index("i")

    barrier = pltpu.get_barrier_semaphore()
    for k in range(1, N):
        nbr = (me + k) % N
        pl.semaphore_signal(barrier, inc=1, device_id=(nbr,),
                            device_id_type=pl.DeviceIdType.MESH)
    pl.semaphore_wait(barrier, N - 1)

    val = val_ref[...]
    flat = flat_ref[0, :]

    send1 = []
    for k in range(1, N):
        dst = (me + k) % N
        xsend_ref[pl.ds(k * C, C), :] = _pack_chunk(k, flat, val)
        r = pltpu.make_async_remote_copy(
            src_ref=xsend_ref.at[pl.ds(k * C, C), :],
            dst_ref=recv_ref.at[pl.ds((N - k) * C, C), :],
            send_sem=ss1.at[k],
            recv_sem=rs1.at[N - k],
            device_id=(dst,),
            device_id_type=pl.DeviceIdType.MESH,
        )
        r.start()
        send1.append(r)

    xsend_ref[pl.ds(0, C), :] = _pack_chunk(0, flat, val)
    ybuf_ref[0] = _expert_chunk(xsend_ref[pl.ds(0, C), :], eW_ref)
    cp = pltpu.make_async_copy(
        ybuf_ref.at[0], yrecv_ref.at[pl.ds(0, C), :], cp_sem)
    cp.start()
    cp.wait()

    send2 = []
    for j in range(1, N):
        s = (me + j) % N
        rwait = pltpu.make_async_remote_copy(
            src_ref=xsend_ref.at[pl.ds(0, C), :],
            dst_ref=recv_ref.at[pl.ds(j * C, C), :],
            send_sem=ss1.at[j],
            recv_sem=rs1.at[j],
            device_id=(s,),
            device_id_type=pl.DeviceIdType.MESH,
        )
        rwait.wait_recv()

        slot = (j - 1) % 2
        if len(send2) >= 2:
            send2[-2].wait_send()
        ybuf_ref[slot] = _expert_chunk(recv_ref[pl.ds(j * C, C), :], eW_ref)

        r = pltpu.make_async_remote_copy(
            src_ref=ybuf_ref.at[slot],
            dst_ref=yrecv_ref.at[pl.ds((N - j) * C, C), :],
            send_sem=ss2.at[j],
            recv_sem=rs2.at[N - j],
            device_id=(s,),
            device_id_type=pl.DeviceIdType.MESH,
        )
        r.start()
        send2.append(r)

    for j in range(1, N):
        rwait = pltpu.make_async_remote_copy(
            src_ref=ybuf_ref.at[0],
            dst_ref=yrecv_ref.at[pl.ds(j * C, C), :],
            send_sem=ss2.at[j],
            recv_sem=rs2.at[j],
            device_id=((me + j) % N,),
            device_id_type=pl.DeviceIdType.MESH,
        )
        rwait.wait_recv()

    for r in send1:
        r.wait_send()
    for r in send2[-2:]:
        r.wait_send()


def kernel(x, router_W, route_idx, expert_W, shared_W):
    me = lax.axis_index("i")

    scores = jnp.dot(x, router_W, preferred_element_type=jnp.float32)
    scores = scores - scores.max(axis=1, keepdims=True)
    probs = jnp.exp(scores)
    probs = probs / probs.sum(axis=1, keepdims=True)
    p = jnp.take_along_axis(probs, route_idx, axis=1)[:, 0]

    e = route_idx[:, 0]
    dst = e // E_LOC
    eloc = e % E_LOC
    rel = (dst - me) % N

    oh_dst = (dst[:, None] == jnp.arange(N, dtype=dst.dtype)[None, :]).astype(jnp.int32)
    j_slot = jnp.take_along_axis(jnp.cumsum(oh_dst, axis=0), dst[:, None], axis=1)[:, 0] - 1
    flat = jnp.where(j_slot < C, rel * C + j_slot, N * C)

    xp = x * p[:, None]
    oh_e = (eloc[:, None] == jnp.arange(E_LOC, dtype=eloc.dtype)[None, :]).astype(x.dtype)
    val = jnp.concatenate(
        [xp, oh_e, jnp.zeros((T, PAY - D - E_LOC), x.dtype)], axis=1)
    flat_f = flat.astype(jnp.int32)[None, :]

    yrecv = pl.pallas_call(
        _body,
        out_shape=jax.ShapeDtypeStruct((N * C, H), jnp.float32),
        in_specs=[
            pl.BlockSpec(memory_space=pltpu.VMEM),
            pl.BlockSpec(memory_space=pltpu.VMEM),
            pl.BlockSpec(memory_space=pltpu.VMEM),
        ],
        out_specs=pl.BlockSpec(memory_space=pl.ANY),
        scratch_shapes=[
            pltpu.VMEM((N * C, PAY), jnp.float32),
            pltpu.VMEM((N * C, PAY), jnp.float32),
            pltpu.VMEM((2, C, H), jnp.float32),
            pltpu.SemaphoreType.DMA((N,)),
            pltpu.SemaphoreType.DMA((N,)),
            pltpu.SemaphoreType.DMA((N,)),
            pltpu.SemaphoreType.DMA((N,)),
            pltpu.SemaphoreType.DMA,
        ],
        compiler_params=pltpu.CompilerParams(
            collective_id=0, vmem_limit_bytes=42 * 1024 * 1024),
    )(val, flat_f, expert_W)

    shared = jnp.dot(x, shared_W, preferred_element_type=jnp.float32)
    g = (flat[:, None] == jnp.arange(N * C, dtype=flat.dtype)[None, :]).astype(x.dtype)
    expert_out = jnp.dot(g, yrecv, preferred_element_type=jnp.float32)
    return shared + expert_out
